# baseline (speedup 1.0000x reference)
import functools

import jax
import jax.numpy as jnp
from jax import lax
from jax.experimental import pallas as pl
from jax.experimental.pallas import tpu as pltpu
from jax.experimental.pallas import tpu_sc as plsc

AVG_NEIGH = 16.0

NC = 2
NS = 16
NW = NC * NS
LANES = 16


def _silu(x):
    return x * (1.0 / (1.0 + jnp.exp(-x)))


def _dot_t(lhs_t, rhs):
    return lax.dot_general(lhs_t, rhs, (((0,), (0,)), ((), ())),
                           preferred_element_type=jnp.float32)


def _node_body(na_ref, nf_ref, wsc_ref, w1_ref, sc_ref, nf2_ref):
    nf = nf_ref[...]
    na = na_ref[...]
    A = na.shape[1]
    tp = jnp.concatenate([na[:, v:v + 1] * nf for v in range(A)], axis=1)
    sc_ref[...] = jnp.dot(tp, wsc_ref[...], preferred_element_type=jnp.float32)
    nf2_ref[...] = jnp.dot(nf, w1_ref[...], preferred_element_type=jnp.float32)


def _node_kernel(node_attrs, node_feats, wsc_v, W1, bn):
    N, D = node_feats.shape
    A = node_attrs.shape[1]
    return pl.pallas_call(
        _node_body,
        grid=(N // bn,),
        in_specs=[
            pl.BlockSpec((bn, A), lambda i: (i, 0)),
            pl.BlockSpec((bn, D), lambda i: (i, 0)),
            pl.BlockSpec((A * D, D), lambda i: (0, 0)),
            pl.BlockSpec((D, D), lambda i: (0, 0)),
        ],
        out_specs=[
            pl.BlockSpec((bn, D), lambda i: (i, 0)),
            pl.BlockSpec((bn, D), lambda i: (i, 0)),
        ],
        out_shape=[
            jax.ShapeDtypeStruct((N, D), jnp.float32),
            jax.ShapeDtypeStruct((N, D), jnp.float32),
        ],
    )(node_attrs, node_feats, wsc_v, W1)


def _edge_body(eft_ref, eat_ref, m0t_ref, m1t_ref, m2t_ref, m3s_ref, w_ref):
    hT = _silu(jnp.dot(m0t_ref[...], eft_ref[...],
                       preferred_element_type=jnp.float32))
    hT = _silu(jnp.dot(m1t_ref[...], hT, preferred_element_type=jnp.float32))
    hT = _silu(jnp.dot(m2t_ref[...], hT, preferred_element_type=jnp.float32))
    SH = eat_ref.shape[0]
    g = jnp.concatenate([hT * eat_ref[v:v + 1, :] for v in range(SH)], axis=0)
    w_ref[...] = _dot_t(g, m3s_ref[...])


def _edge_kernel(ef_t, ea_t, m0t, m1t, m2t, m3_s, be, e0, e_seg):
    RB, E = ef_t.shape
    SH = ea_t.shape[0]
    HID = m0t.shape[0]
    D = m3_s.shape[1]
    off = e0 // be
    return pl.pallas_call(
        _edge_body,
        grid=(e_seg // be,),
        in_specs=[
            pl.BlockSpec((RB, be), lambda i: (0, i + off)),
            pl.BlockSpec((SH, be), lambda i: (0, i + off)),
            pl.BlockSpec((HID, RB), lambda i: (0, 0)),
            pl.BlockSpec((HID, HID), lambda i: (0, 0)),
            pl.BlockSpec((HID, HID), lambda i: (0, 0)),
            pl.BlockSpec((SH * HID, D), lambda i: (0, 0)),
        ],
        out_specs=pl.BlockSpec((be, D), lambda i: (i, 0)),
        out_shape=jax.ShapeDtypeStruct((e_seg, D), jnp.float32),
    )(ef_t, ea_t, m0t, m1t, m2t, m3_s)


def _make_sc_kernel(N, D, E, C):
    n_chunks = E // C
    per = n_chunks // NW
    extra = n_chunks - per * NW
    iters = per + 1
    ZR = (N // NS) // 8 * 8
    TAIL = N - ZR * NS

    mesh = plsc.VectorSubcoreMesh(
        core_axis_name="c", subcore_axis_name="s",
        num_cores=NC, num_subcores=NS)

    @functools.partial(
        pl.kernel,
        out_type=jax.ShapeDtypeStruct((NC, N, D), jnp.float32),
        mesh=mesh,
        scratch_types=[
            [pltpu.VMEM((C,), jnp.int32) for _ in range(2)],
            [pltpu.VMEM((C,), jnp.int32) for _ in range(2)],
            [pltpu.VMEM((C,), jnp.int32) for _ in range(2)],
            [pltpu.VMEM((C, D), jnp.float32) for _ in range(2)],
            [pltpu.VMEM((C, D), jnp.float32) for _ in range(2)],
            [pltpu.VMEM((C, D), jnp.float32) for _ in range(2)],
            pltpu.VMEM_SHARED((N, D), jnp.float32),
            [pltpu.SemaphoreType.DMA for _ in range(2)],
            [pltpu.SemaphoreType.DMA for _ in range(2)],
            [pltpu.SemaphoreType.DMA for _ in range(2)],
            [pltpu.SemaphoreType.DMA for _ in range(2)],
        ],
    )
    def sc_kernel(nf_hbm, w_hbm, zeros_hbm, snd_hbm, rcv_hbm, out_hbm,
                  sidx_ld, ridx_ld, ridx_sc, rows, wrows, prod, msg_sh,
                  isem, gsem, wsem, ssem):
        cid = lax.axis_index("c")
        sid = lax.axis_index("s")
        wid = sid * NC + cid
        start = wid * per + lax.min(wid, extra)
        count = per + jnp.where(wid < extra, 1, 0)

        def idx_load(i, b):
            @pl.when(i < count)
            def _():
                base = pl.multiple_of((start + i) * C, 8)
                pltpu.async_copy(snd_hbm.at[pl.ds(base, C)], sidx_ld[b],
                                 isem[b])
                pltpu.async_copy(rcv_hbm.at[pl.ds(base, C)], ridx_ld[b],
                                 isem[b])

        def idx_wait(i, b):
            @pl.when(i < count)
            def _():
                pltpu.make_async_copy(snd_hbm.at[pl.ds(0, C)], sidx_ld[b],
                                      isem[b]).wait()
                pltpu.make_async_copy(snd_hbm.at[pl.ds(0, C)], sidx_ld[b],
                                      isem[b]).wait()

        def fetch(i, b):
            @pl.when(i < count)
            def _():
                base = pl.multiple_of((start + i) * C, 8)
                pltpu.async_copy(nf_hbm.at[sidx_ld[b]], rows[b], gsem[b])
                pltpu.async_copy(w_hbm.at[pl.ds(base, C)], wrows[b], wsem[b])

        idx_load(jnp.int32(0), 0)
        idx_load(jnp.int32(1), 1)

        r0 = pl.multiple_of(sid * ZR, 8)
        pltpu.sync_copy(zeros_hbm.at[pl.ds(r0, ZR)],
                        msg_sh.at[pl.ds(r0, ZR)])
        if TAIL:
            @pl.when(sid == NS - 1)
            def _zero_tail():
                pltpu.sync_copy(zeros_hbm.at[pl.ds(ZR * NS, TAIL)],
                                msg_sh.at[pl.ds(ZR * NS, TAIL)])
        plsc.subcore_barrier()

        idx_wait(jnp.int32(0), 0)
        fetch(jnp.int32(0), 0)

        def step(i, b):
            @pl.when(i < count)
            def _():
                @pl.when(i >= 2)
                def _wait_prev_scatter():
                    pltpu.make_async_copy(
                        prod[b], msg_sh.at[ridx_sc[b]], ssem[b]).wait()

                pltpu.make_async_copy(nf_hbm.at[sidx_ld[b]], rows[b],
                                      gsem[b]).wait()
                pltpu.make_async_copy(w_hbm.at[pl.ds(0, C)], wrows[b],
                                      wsem[b]).wait()

                for j in range(C // LANES):
                    sl = pl.ds(j * LANES, LANES)
                    ridx_sc[b][sl] = ridx_ld[b][sl]

                idx_wait(i + 1, 1 - b)
                fetch(i + 1, 1 - b)
                idx_load(i + 2, b)

                def mul_body(r, _):
                    for j in range(D // LANES):
                        sl = pl.ds(j * LANES, LANES)
                        prod[b][r, sl] = rows[b][r, sl] * wrows[b][r, sl]
                    return 0

                lax.fori_loop(0, C, mul_body, 0)

                pltpu.async_copy(prod[b], msg_sh.at[ridx_sc[b]], ssem[b],
                                 add=True)

        def pair(j, _):
            step(2 * j, 0)
            step(2 * j + 1, 1)
            return 0

        lax.fori_loop(0, (iters + 1) // 2, pair, 0)

        for b in range(2):
            pltpu.make_async_copy(prod[b], msg_sh.at[ridx_sc[b]],
                                  ssem[b]).wait()

        plsc.subcore_barrier()
        pltpu.sync_copy(msg_sh.at[pl.ds(r0, ZR)],
                        out_hbm.at[cid, pl.ds(r0, ZR)])
        if TAIL:
            @pl.when(sid == NS - 1)
            def _write_tail():
                pltpu.sync_copy(msg_sh.at[pl.ds(ZR * NS, TAIL)],
                                out_hbm.at[cid, pl.ds(ZR * NS, TAIL)])

    return sc_kernel


def _final_body(*refs):
    p_refs = refs[:-2]
    w2_ref = refs[-2]
    out_ref = refs[-1]
    m = p_refs[0][0] + p_refs[0][1]
    for p in p_refs[1:]:
        m = m + p[0] + p[1]
    out_ref[...] = jnp.dot(
        m, w2_ref[...], preferred_element_type=jnp.float32) * (1.0 / AVG_NEIGH)


def _final_kernel(partials_list, W2, bn):
    _, N, D = partials_list[0].shape
    return pl.pallas_call(
        _final_body,
        grid=(N // bn,),
        in_specs=(
            [pl.BlockSpec((NC, bn, D), lambda i: (0, i, 0))
             for _ in partials_list]
            + [pl.BlockSpec((D, D), lambda i: (0, 0))]),
        out_specs=pl.BlockSpec((bn, D), lambda i: (i, 0)),
        out_shape=jax.ShapeDtypeStruct((N, D), jnp.float32),
    )(*partials_list, W2)


def kernel(node_attrs, node_feats, edge_attrs, edge_feats, senders, receivers,
           W_sc, W1, M0, M1, M2, M3, W2):
    N, D = node_feats.shape
    A = node_attrs.shape[1]
    E, SH = edge_attrs.shape
    HID = M0.shape[1]

    wsc_v = jnp.transpose(W_sc.reshape(D, A, D), (1, 0, 2)).reshape(A * D, D)
    m3_s = jnp.transpose(M3.reshape(HID, D, SH), (2, 0, 1)).reshape(SH * HID, D)
    snd = senders.astype(jnp.int32)
    rcv = receivers.astype(jnp.int32)

    sc, nf2 = _node_kernel(node_attrs, node_feats, wsc_v, W1, bn=1000)

    segs = (57600, E - 57600)
    zeros = jnp.zeros((N, D), jnp.float32)
    ef_t = edge_feats.T
    ea_t = edge_attrs.T
    partials_list = []
    e0 = 0
    for e_seg in segs:
        w_s = _edge_kernel(ef_t, ea_t, M0.T, M1.T, M2.T, m3_s,
                           be=3200, e0=e0, e_seg=e_seg)
        partials_list.append(
            _make_sc_kernel(N, D, e_seg, C=64)(
                nf2, w_s, zeros, snd[e0:e0 + e_seg], rcv[e0:e0 + e_seg]))
        e0 += e_seg

    out = _final_kernel(partials_list, W2, bn=1000)
    return (out, sc)


# speedup vs baseline: 1.0492x; 1.0492x over previous
import functools

import jax
import jax.numpy as jnp
from jax import lax
from jax.experimental import pallas as pl
from jax.experimental.pallas import tpu as pltpu
from jax.experimental.pallas import tpu_sc as plsc

AVG_NEIGH = 16.0

NC = 2
NS = 16
NW = NC * NS
LANES = 16


def _silu(x):
    return x * (1.0 / (1.0 + jnp.exp(-x)))


def _dot_t(lhs_t, rhs):
    return lax.dot_general(lhs_t, rhs, (((0,), (0,)), ((), ())),
                           preferred_element_type=jnp.float32)


def _node_body(na_ref, nf_ref, wsc_ref, w1_ref, sc_ref, nf2_ref):
    nf = nf_ref[...]
    na = na_ref[...]
    A = na.shape[1]
    tp = jnp.concatenate([na[:, v:v + 1] * nf for v in range(A)], axis=1)
    sc_ref[...] = jnp.dot(tp, wsc_ref[...], preferred_element_type=jnp.float32)
    nf2_ref[...] = jnp.dot(nf, w1_ref[...], preferred_element_type=jnp.float32)


def _node_kernel(node_attrs, node_feats, wsc_v, W1, bn):
    N, D = node_feats.shape
    A = node_attrs.shape[1]
    return pl.pallas_call(
        _node_body,
        grid=(N // bn,),
        in_specs=[
            pl.BlockSpec((bn, A), lambda i: (i, 0)),
            pl.BlockSpec((bn, D), lambda i: (i, 0)),
            pl.BlockSpec((A * D, D), lambda i: (0, 0)),
            pl.BlockSpec((D, D), lambda i: (0, 0)),
        ],
        out_specs=[
            pl.BlockSpec((bn, D), lambda i: (i, 0)),
            pl.BlockSpec((bn, D), lambda i: (i, 0)),
        ],
        out_shape=[
            jax.ShapeDtypeStruct((N, D), jnp.float32),
            jax.ShapeDtypeStruct((N, D), jnp.float32),
        ],
    )(node_attrs, node_feats, wsc_v, W1)


def _edge_body(eft_ref, eat_ref, m0t_ref, m1t_ref, m2t_ref, m3s_ref, w_ref):
    hT = _silu(jnp.dot(m0t_ref[...], eft_ref[...],
                       preferred_element_type=jnp.float32))
    hT = _silu(jnp.dot(m1t_ref[...], hT, preferred_element_type=jnp.float32))
    hT = _silu(jnp.dot(m2t_ref[...], hT, preferred_element_type=jnp.float32))
    SH = eat_ref.shape[0]
    g = jnp.concatenate([hT * eat_ref[v:v + 1, :] for v in range(SH)], axis=0)
    w_ref[...] = _dot_t(g, m3s_ref[...])


def _edge_kernel(ef_t, ea_t, m0t, m1t, m2t, m3_s, be, e0, e_seg):
    RB, E = ef_t.shape
    SH = ea_t.shape[0]
    HID = m0t.shape[0]
    D = m3_s.shape[1]
    off = e0 // be
    return pl.pallas_call(
        _edge_body,
        grid=(e_seg // be,),
        in_specs=[
            pl.BlockSpec((RB, be), lambda i: (0, i + off)),
            pl.BlockSpec((SH, be), lambda i: (0, i + off)),
            pl.BlockSpec((HID, RB), lambda i: (0, 0)),
            pl.BlockSpec((HID, HID), lambda i: (0, 0)),
            pl.BlockSpec((HID, HID), lambda i: (0, 0)),
            pl.BlockSpec((SH * HID, D), lambda i: (0, 0)),
        ],
        out_specs=pl.BlockSpec((be, D), lambda i: (i, 0)),
        out_shape=jax.ShapeDtypeStruct((e_seg, D), jnp.float32),
    )(ef_t, ea_t, m0t, m1t, m2t, m3_s)


def _make_sc_kernel(N, D, E, C):
    n_chunks = E // C
    per = n_chunks // NW
    extra = n_chunks - per * NW
    iters = per + 1
    ZR = (N // NS) // 8 * 8
    TAIL = N - ZR * NS

    mesh = plsc.VectorSubcoreMesh(
        core_axis_name="c", subcore_axis_name="s",
        num_cores=NC, num_subcores=NS)

    @functools.partial(
        pl.kernel,
        out_type=jax.ShapeDtypeStruct((NC, N, D), jnp.float32),
        mesh=mesh,
        scratch_types=[
            [pltpu.VMEM((C,), jnp.int32) for _ in range(2)],
            [pltpu.VMEM((C,), jnp.int32) for _ in range(2)],
            [pltpu.VMEM((C,), jnp.int32) for _ in range(2)],
            [pltpu.VMEM((C, D), jnp.float32) for _ in range(2)],
            [pltpu.VMEM((C, D), jnp.float32) for _ in range(2)],
            [pltpu.VMEM((C, D), jnp.float32) for _ in range(2)],
            pltpu.VMEM_SHARED((N, D), jnp.float32),
            [pltpu.SemaphoreType.DMA for _ in range(2)],
            [pltpu.SemaphoreType.DMA for _ in range(2)],
            [pltpu.SemaphoreType.DMA for _ in range(2)],
            [pltpu.SemaphoreType.DMA for _ in range(2)],
        ],
    )
    def sc_kernel(nf_hbm, w_hbm, init_hbm, snd_hbm, rcv_hbm, out_hbm,
                  sidx_ld, ridx_ld, ridx_sc, rows, wrows, prod, msg_sh,
                  isem, gsem, wsem, ssem):
        cid = lax.axis_index("c")
        sid = lax.axis_index("s")
        wid = sid * NC + cid
        start = wid * per + lax.min(wid, extra)
        count = per + jnp.where(wid < extra, 1, 0)
        init2d = init_hbm.at[cid] if init_hbm.shape == (NC, N, D) else init_hbm

        def idx_load(i, b):
            @pl.when(i < count)
            def _():
                base = pl.multiple_of((start + i) * C, 8)
                pltpu.async_copy(snd_hbm.at[pl.ds(base, C)], sidx_ld[b],
                                 isem[b])
                pltpu.async_copy(rcv_hbm.at[pl.ds(base, C)], ridx_ld[b],
                                 isem[b])

        def idx_wait(i, b):
            @pl.when(i < count)
            def _():
                pltpu.make_async_copy(snd_hbm.at[pl.ds(0, C)], sidx_ld[b],
                                      isem[b]).wait()
                pltpu.make_async_copy(snd_hbm.at[pl.ds(0, C)], sidx_ld[b],
                                      isem[b]).wait()

        def fetch(i, b):
            @pl.when(i < count)
            def _():
                base = pl.multiple_of((start + i) * C, 8)
                pltpu.async_copy(nf_hbm.at[sidx_ld[b]], rows[b], gsem[b])
                pltpu.async_copy(w_hbm.at[pl.ds(base, C)], wrows[b], wsem[b])

        idx_load(jnp.int32(0), 0)
        idx_load(jnp.int32(1), 1)

        r0 = pl.multiple_of(sid * ZR, 8)
        pltpu.sync_copy(init2d.at[pl.ds(r0, ZR)],
                        msg_sh.at[pl.ds(r0, ZR)])
        if TAIL:
            @pl.when(sid == NS - 1)
            def _init_tail():
                pltpu.sync_copy(init2d.at[pl.ds(ZR * NS, TAIL)],
                                msg_sh.at[pl.ds(ZR * NS, TAIL)])
        plsc.subcore_barrier()

        idx_wait(jnp.int32(0), 0)
        fetch(jnp.int32(0), 0)

        def step(i, b):
            @pl.when(i < count)
            def _():
                @pl.when(i >= 2)
                def _wait_prev_scatter():
                    pltpu.make_async_copy(
                        prod[b], msg_sh.at[ridx_sc[b]], ssem[b]).wait()

                pltpu.make_async_copy(nf_hbm.at[sidx_ld[b]], rows[b],
                                      gsem[b]).wait()
                pltpu.make_async_copy(w_hbm.at[pl.ds(0, C)], wrows[b],
                                      wsem[b]).wait()

                for j in range(C // LANES):
                    sl = pl.ds(j * LANES, LANES)
                    ridx_sc[b][sl] = ridx_ld[b][sl]

                idx_wait(i + 1, 1 - b)
                fetch(i + 1, 1 - b)
                idx_load(i + 2, b)

                def mul_body(r, _):
                    for j in range(D // LANES):
                        sl = pl.ds(j * LANES, LANES)
                        prod[b][r, sl] = rows[b][r, sl] * wrows[b][r, sl]
                    return 0

                lax.fori_loop(0, C, mul_body, 0)

                pltpu.async_copy(prod[b], msg_sh.at[ridx_sc[b]], ssem[b],
                                 add=True)

        def pair(j, _):
            step(2 * j, 0)
            step(2 * j + 1, 1)
            return 0

        lax.fori_loop(0, (iters + 1) // 2, pair, 0)

        for b in range(2):
            pltpu.make_async_copy(prod[b], msg_sh.at[ridx_sc[b]],
                                  ssem[b]).wait()

        plsc.subcore_barrier()
        pltpu.sync_copy(msg_sh.at[pl.ds(r0, ZR)],
                        out_hbm.at[cid, pl.ds(r0, ZR)])
        if TAIL:
            @pl.when(sid == NS - 1)
            def _write_tail():
                pltpu.sync_copy(msg_sh.at[pl.ds(ZR * NS, TAIL)],
                                out_hbm.at[cid, pl.ds(ZR * NS, TAIL)])

    return sc_kernel


def _final_body(*refs):
    p_refs = refs[:-2]
    w2_ref = refs[-2]
    out_ref = refs[-1]
    m = p_refs[0][0] + p_refs[0][1]
    for p in p_refs[1:]:
        m = m + p[0] + p[1]
    out_ref[...] = jnp.dot(
        m, w2_ref[...], preferred_element_type=jnp.float32) * (1.0 / AVG_NEIGH)


def _final_kernel(partials_list, W2, bn):
    _, N, D = partials_list[0].shape
    return pl.pallas_call(
        _final_body,
        grid=(N // bn,),
        in_specs=(
            [pl.BlockSpec((NC, bn, D), lambda i: (0, i, 0))
             for _ in partials_list]
            + [pl.BlockSpec((D, D), lambda i: (0, 0))]),
        out_specs=pl.BlockSpec((bn, D), lambda i: (i, 0)),
        out_shape=jax.ShapeDtypeStruct((N, D), jnp.float32),
    )(*partials_list, W2)


def kernel(node_attrs, node_feats, edge_attrs, edge_feats, senders, receivers,
           W_sc, W1, M0, M1, M2, M3, W2):
    N, D = node_feats.shape
    A = node_attrs.shape[1]
    E, SH = edge_attrs.shape
    HID = M0.shape[1]

    wsc_v = jnp.transpose(W_sc.reshape(D, A, D), (1, 0, 2)).reshape(A * D, D)
    m3_s = jnp.transpose(M3.reshape(HID, D, SH), (2, 0, 1)).reshape(SH * HID, D)
    snd = senders.astype(jnp.int32)
    rcv = receivers.astype(jnp.int32)

    sc, nf2 = _node_kernel(node_attrs, node_feats, wsc_v, W1, bn=1000)

    segs = (E // 2, E // 2)
    init = jnp.zeros((N, D), jnp.float32)
    ef_t = edge_feats.T
    ea_t = edge_attrs.T
    e0 = 0
    for e_seg in segs:
        w_s = _edge_kernel(ef_t, ea_t, M0.T, M1.T, M2.T, m3_s,
                           be=3200, e0=e0, e_seg=e_seg)
        init = _make_sc_kernel(N, D, e_seg, C=64)(
            nf2, w_s, init, snd[e0:e0 + e_seg], rcv[e0:e0 + e_seg])
        e0 += e_seg

    out = _final_kernel([init], W2, bn=1000)
    return (out, sc)


# speedup vs baseline: 1.0844x; 1.0335x over previous
import functools

import jax
import jax.numpy as jnp
from jax import lax
from jax.experimental import pallas as pl
from jax.experimental.pallas import tpu as pltpu
from jax.experimental.pallas import tpu_sc as plsc

AVG_NEIGH = 16.0

NC = 2
NS = 16
NW = NC * NS
LANES = 16


def _silu(x):
    return x * (1.0 / (1.0 + jnp.exp(-x)))


def _dot_t(lhs_t, rhs):
    return lax.dot_general(lhs_t, rhs, (((0,), (0,)), ((), ())),
                           preferred_element_type=jnp.float32)


def _nf2_body(nf_ref, w1_ref, nf2_ref):
    nf2_ref[...] = jnp.dot(nf_ref[...], w1_ref[...],
                           preferred_element_type=jnp.float32)


def _nf2_kernel(node_feats, W1, bn):
    N, D = node_feats.shape
    return pl.pallas_call(
        _nf2_body,
        grid=(N // bn,),
        in_specs=[
            pl.BlockSpec((bn, D), lambda i: (i, 0)),
            pl.BlockSpec((D, D), lambda i: (0, 0)),
        ],
        out_specs=pl.BlockSpec((bn, D), lambda i: (i, 0)),
        out_shape=jax.ShapeDtypeStruct((N, D), jnp.float32),
    )(node_feats, W1)


def _sc_body(na_ref, nf_ref, wsc_ref, sc_ref):
    nf = nf_ref[...]
    na = na_ref[...]
    A = na.shape[1]
    tp = jnp.concatenate([na[:, v:v + 1] * nf for v in range(A)], axis=1)
    sc_ref[...] = jnp.dot(tp, wsc_ref[...], preferred_element_type=jnp.float32)


def _sc_kernel(node_attrs, node_feats, wsc_v, bn):
    N, D = node_feats.shape
    A = node_attrs.shape[1]
    return pl.pallas_call(
        _sc_body,
        grid=(N // bn,),
        in_specs=[
            pl.BlockSpec((bn, A), lambda i: (i, 0)),
            pl.BlockSpec((bn, D), lambda i: (i, 0)),
            pl.BlockSpec((A * D, D), lambda i: (0, 0)),
        ],
        out_specs=pl.BlockSpec((bn, D), lambda i: (i, 0)),
        out_shape=jax.ShapeDtypeStruct((N, D), jnp.float32),
    )(node_attrs, node_feats, wsc_v)


def _edge_body(eft_ref, eat_ref, m0t_ref, m1t_ref, m2t_ref, m3s_ref, w_ref):
    hT = _silu(jnp.dot(m0t_ref[...], eft_ref[...],
                       preferred_element_type=jnp.float32))
    hT = _silu(jnp.dot(m1t_ref[...], hT, preferred_element_type=jnp.float32))
    hT = _silu(jnp.dot(m2t_ref[...], hT, preferred_element_type=jnp.float32))
    SH = eat_ref.shape[0]
    g = jnp.concatenate([hT * eat_ref[v:v + 1, :] for v in range(SH)], axis=0)
    w_ref[...] = _dot_t(g, m3s_ref[...])


def _edge_kernel(ef_t, ea_t, m0t, m1t, m2t, m3_s, be, e0, e_seg):
    RB, E = ef_t.shape
    SH = ea_t.shape[0]
    HID = m0t.shape[0]
    D = m3_s.shape[1]
    off = e0 // be
    return pl.pallas_call(
        _edge_body,
        grid=(e_seg // be,),
        in_specs=[
            pl.BlockSpec((RB, be), lambda i: (0, i + off)),
            pl.BlockSpec((SH, be), lambda i: (0, i + off)),
            pl.BlockSpec((HID, RB), lambda i: (0, 0)),
            pl.BlockSpec((HID, HID), lambda i: (0, 0)),
            pl.BlockSpec((HID, HID), lambda i: (0, 0)),
            pl.BlockSpec((SH * HID, D), lambda i: (0, 0)),
        ],
        out_specs=pl.BlockSpec((be, D), lambda i: (i, 0)),
        out_shape=jax.ShapeDtypeStruct((e_seg, D), jnp.float32),
    )(ef_t, ea_t, m0t, m1t, m2t, m3_s)


def _make_sc_kernel(N, D, E, C, e0):
    n_chunks = E // C
    per = n_chunks // NW
    extra = n_chunks - per * NW
    iters = per + 1
    ZR = (N // NS) // 8 * 8
    TAIL = N - ZR * NS

    mesh = plsc.VectorSubcoreMesh(
        core_axis_name="c", subcore_axis_name="s",
        num_cores=NC, num_subcores=NS)

    @functools.partial(
        pl.kernel,
        out_type=jax.ShapeDtypeStruct((NC, N, D), jnp.float32),
        mesh=mesh,
        scratch_types=[
            [pltpu.VMEM((C,), jnp.int32) for _ in range(2)],
            [pltpu.VMEM((C,), jnp.int32) for _ in range(2)],
            [pltpu.VMEM((C,), jnp.int32) for _ in range(2)],
            [pltpu.VMEM((C, D), jnp.float32) for _ in range(2)],
            [pltpu.VMEM((C, D), jnp.float32) for _ in range(2)],
            [pltpu.VMEM((C, D), jnp.float32) for _ in range(2)],
            pltpu.VMEM_SHARED((N, D), jnp.float32),
            [pltpu.SemaphoreType.DMA for _ in range(2)],
            [pltpu.SemaphoreType.DMA for _ in range(2)],
            [pltpu.SemaphoreType.DMA for _ in range(2)],
            [pltpu.SemaphoreType.DMA for _ in range(2)],
        ],
    )
    def sc_kernel(nf_hbm, w_hbm, init_hbm, snd_hbm, rcv_hbm, out_hbm,
                  sidx_ld, ridx_ld, ridx_sc, rows, wrows, prod, msg_sh,
                  isem, gsem, wsem, ssem):
        cid = lax.axis_index("c")
        sid = lax.axis_index("s")
        wid = sid * NC + cid
        start = wid * per + lax.min(wid, extra)
        count = per + jnp.where(wid < extra, 1, 0)
        init2d = init_hbm.at[cid] if init_hbm.shape == (NC, N, D) else init_hbm

        def idx_load(i, b):
            @pl.when(i < count)
            def _():
                base = pl.multiple_of(e0 + (start + i) * C, 8)
                pltpu.async_copy(snd_hbm.at[pl.ds(base, C)], sidx_ld[b],
                                 isem[b])
                pltpu.async_copy(rcv_hbm.at[pl.ds(base, C)], ridx_ld[b],
                                 isem[b])

        def idx_wait(i, b):
            @pl.when(i < count)
            def _():
                pltpu.make_async_copy(snd_hbm.at[pl.ds(0, C)], sidx_ld[b],
                                      isem[b]).wait()
                pltpu.make_async_copy(snd_hbm.at[pl.ds(0, C)], sidx_ld[b],
                                      isem[b]).wait()

        def fetch(i, b):
            @pl.when(i < count)
            def _():
                base = pl.multiple_of((start + i) * C, 8)
                pltpu.async_copy(nf_hbm.at[sidx_ld[b]], rows[b], gsem[b])
                pltpu.async_copy(w_hbm.at[pl.ds(base, C)], wrows[b], wsem[b])

        idx_load(jnp.int32(0), 0)
        idx_load(jnp.int32(1), 1)

        r0 = pl.multiple_of(sid * ZR, 8)
        pltpu.sync_copy(init2d.at[pl.ds(r0, ZR)],
                        msg_sh.at[pl.ds(r0, ZR)])
        if TAIL:
            @pl.when(sid == NS - 1)
            def _init_tail():
                pltpu.sync_copy(init2d.at[pl.ds(ZR * NS, TAIL)],
                                msg_sh.at[pl.ds(ZR * NS, TAIL)])
        plsc.subcore_barrier()

        idx_wait(jnp.int32(0), 0)
        fetch(jnp.int32(0), 0)

        def step(i, b):
            @pl.when(i < count)
            def _():
                @pl.when(i >= 2)
                def _wait_prev_scatter():
                    pltpu.make_async_copy(
                        prod[b], msg_sh.at[ridx_sc[b]], ssem[b]).wait()

                pltpu.make_async_copy(nf_hbm.at[sidx_ld[b]], rows[b],
                                      gsem[b]).wait()
                pltpu.make_async_copy(w_hbm.at[pl.ds(0, C)], wrows[b],
                                      wsem[b]).wait()

                for j in range(C // LANES):
                    sl = pl.ds(j * LANES, LANES)
                    ridx_sc[b][sl] = ridx_ld[b][sl]

                idx_wait(i + 1, 1 - b)
                fetch(i + 1, 1 - b)
                idx_load(i + 2, b)

                @plsc.parallel_loop(0, C, unroll=4)
                def _mul(r):
                    for j in range(D // LANES):
                        sl = pl.ds(j * LANES, LANES)
                        prod[b][r, sl] = rows[b][r, sl] * wrows[b][r, sl]

                pltpu.async_copy(prod[b], msg_sh.at[ridx_sc[b]], ssem[b],
                                 add=True)

        def pair(j, _):
            step(2 * j, 0)
            step(2 * j + 1, 1)
            return 0

        lax.fori_loop(0, (iters + 1) // 2, pair, 0)

        for b in range(2):
            pltpu.make_async_copy(prod[b], msg_sh.at[ridx_sc[b]],
                                  ssem[b]).wait()

        plsc.subcore_barrier()
        pltpu.sync_copy(msg_sh.at[pl.ds(r0, ZR)],
                        out_hbm.at[cid, pl.ds(r0, ZR)])
        if TAIL:
            @pl.when(sid == NS - 1)
            def _write_tail():
                pltpu.sync_copy(msg_sh.at[pl.ds(ZR * NS, TAIL)],
                                out_hbm.at[cid, pl.ds(ZR * NS, TAIL)])

    return sc_kernel


def _final_body(*refs):
    p_refs = refs[:-2]
    w2_ref = refs[-2]
    out_ref = refs[-1]
    m = p_refs[0][0] + p_refs[0][1]
    for p in p_refs[1:]:
        m = m + p[0] + p[1]
    out_ref[...] = jnp.dot(
        m, w2_ref[...], preferred_element_type=jnp.float32) * (1.0 / AVG_NEIGH)


def _final_kernel(partials_list, W2, bn):
    _, N, D = partials_list[0].shape
    return pl.pallas_call(
        _final_body,
        grid=(N // bn,),
        in_specs=(
            [pl.BlockSpec((NC, bn, D), lambda i: (0, i, 0))
             for _ in partials_list]
            + [pl.BlockSpec((D, D), lambda i: (0, 0))]),
        out_specs=pl.BlockSpec((bn, D), lambda i: (i, 0)),
        out_shape=jax.ShapeDtypeStruct((N, D), jnp.float32),
    )(*partials_list, W2)


def kernel(node_attrs, node_feats, edge_attrs, edge_feats, senders, receivers,
           W_sc, W1, M0, M1, M2, M3, W2):
    N, D = node_feats.shape
    A = node_attrs.shape[1]
    E, SH = edge_attrs.shape
    HID = M0.shape[1]

    wsc_v = jnp.transpose(W_sc.reshape(D, A, D), (1, 0, 2)).reshape(A * D, D)
    m3_s = jnp.transpose(M3.reshape(HID, D, SH), (2, 0, 1)).reshape(SH * HID, D)
    snd = senders.astype(jnp.int32)
    rcv = receivers.astype(jnp.int32)

    nf2 = _nf2_kernel(node_feats, W1, bn=1000)

    segs = (E // 2, E // 2)
    init = jnp.zeros((N, D), jnp.float32)
    ef_t = edge_feats.T
    ea_t = edge_attrs.T
    e0 = 0
    for e_seg in segs:
        w_s = _edge_kernel(ef_t, ea_t, M0.T, M1.T, M2.T, m3_s,
                           be=3200, e0=e0, e_seg=e_seg)
        init = _make_sc_kernel(N, D, e_seg, C=64, e0=e0)(
            nf2, w_s, init, snd, rcv)
        e0 += e_seg

    sc = _sc_kernel(node_attrs, node_feats, wsc_v, bn=1000)
    out = _final_kernel([init], W2, bn=1000)
    return (out, sc)


# speedup vs baseline: 1.1200x; 1.0328x over previous
import functools

import jax
import jax.numpy as jnp
from jax import lax
from jax.experimental import pallas as pl
from jax.experimental.pallas import tpu as pltpu
from jax.experimental.pallas import tpu_sc as plsc

AVG_NEIGH = 16.0

NC = 2
NS = 16
NW = NC * NS
LANES = 16


def _silu(x):
    return x * (1.0 / (1.0 + jnp.exp(-x)))


def _dot_t(lhs_t, rhs):
    return lax.dot_general(lhs_t, rhs, (((0,), (0,)), ((), ())),
                           preferred_element_type=jnp.float32)


def _nf2_body(nf_ref, w1_ref, nf2_ref):
    nf2_ref[...] = jnp.dot(nf_ref[...], w1_ref[...],
                           preferred_element_type=jnp.float32)


def _nf2_kernel(node_feats, W1, bn):
    N, D = node_feats.shape
    return pl.pallas_call(
        _nf2_body,
        grid=(N // bn,),
        in_specs=[
            pl.BlockSpec((bn, D), lambda i: (i, 0)),
            pl.BlockSpec((D, D), lambda i: (0, 0)),
        ],
        out_specs=pl.BlockSpec((bn, D), lambda i: (i, 0)),
        out_shape=jax.ShapeDtypeStruct((N, D), jnp.float32),
    )(node_feats, W1)


def _sc_body(na_ref, nf_ref, wsc_ref, sc_ref):
    nf = nf_ref[...]
    na = na_ref[...]
    A = na.shape[1]
    tp = jnp.concatenate([na[:, v:v + 1] * nf for v in range(A)], axis=1)
    sc_ref[...] = jnp.dot(tp, wsc_ref[...], preferred_element_type=jnp.float32)


def _sc_kernel(node_attrs, node_feats, wsc_v, bn):
    N, D = node_feats.shape
    A = node_attrs.shape[1]
    return pl.pallas_call(
        _sc_body,
        grid=(N // bn,),
        in_specs=[
            pl.BlockSpec((bn, A), lambda i: (i, 0)),
            pl.BlockSpec((bn, D), lambda i: (i, 0)),
            pl.BlockSpec((A * D, D), lambda i: (0, 0)),
        ],
        out_specs=pl.BlockSpec((bn, D), lambda i: (i, 0)),
        out_shape=jax.ShapeDtypeStruct((N, D), jnp.float32),
    )(node_attrs, node_feats, wsc_v)


def _edge_body(eft_ref, eat_ref, m0t_ref, m1t_ref, m2t_ref, m3s_ref, w_ref):
    hT = _silu(jnp.dot(m0t_ref[...], eft_ref[...],
                       preferred_element_type=jnp.float32))
    hT = _silu(jnp.dot(m1t_ref[...], hT, preferred_element_type=jnp.float32))
    hT = _silu(jnp.dot(m2t_ref[...], hT, preferred_element_type=jnp.float32))
    SH = eat_ref.shape[0]
    g = jnp.concatenate([hT * eat_ref[v:v + 1, :] for v in range(SH)], axis=0)
    w_ref[...] = _dot_t(g.astype(jnp.bfloat16), m3s_ref[...])


def _edge_kernel(ef_t, ea_t, m0t, m1t, m2t, m3_s, be, e0, e_seg):
    RB, E = ef_t.shape
    SH = ea_t.shape[0]
    HID = m0t.shape[0]
    D = m3_s.shape[1]
    off = e0 // be
    return pl.pallas_call(
        _edge_body,
        grid=(e_seg // be,),
        in_specs=[
            pl.BlockSpec((RB, be), lambda i: (0, i + off)),
            pl.BlockSpec((SH, be), lambda i: (0, i + off)),
            pl.BlockSpec((HID, RB), lambda i: (0, 0)),
            pl.BlockSpec((HID, HID), lambda i: (0, 0)),
            pl.BlockSpec((HID, HID), lambda i: (0, 0)),
            pl.BlockSpec((SH * HID, D), lambda i: (0, 0)),
        ],
        out_specs=pl.BlockSpec((be, D), lambda i: (i, 0)),
        out_shape=jax.ShapeDtypeStruct((e_seg, D), jnp.float32),
    )(ef_t, ea_t, m0t, m1t, m2t, m3_s)


def _make_sc_kernel(N, D, E, C, e0):
    n_chunks = E // C
    per = n_chunks // NW
    extra = n_chunks - per * NW
    iters = per + 1
    ZR = (N // NS) // 8 * 8
    TAIL = N - ZR * NS

    mesh = plsc.VectorSubcoreMesh(
        core_axis_name="c", subcore_axis_name="s",
        num_cores=NC, num_subcores=NS)

    @functools.partial(
        pl.kernel,
        out_type=jax.ShapeDtypeStruct((NC, N, D), jnp.float32),
        mesh=mesh,
        scratch_types=[
            [pltpu.VMEM((C,), jnp.int32) for _ in range(2)],
            [pltpu.VMEM((C,), jnp.int32) for _ in range(2)],
            [pltpu.VMEM((C,), jnp.int32) for _ in range(2)],
            [pltpu.VMEM((C, D), jnp.float32) for _ in range(2)],
            [pltpu.VMEM((C, D), jnp.float32) for _ in range(2)],
            [pltpu.VMEM((C, D), jnp.float32) for _ in range(2)],
            pltpu.VMEM_SHARED((N, D), jnp.float32),
            [pltpu.SemaphoreType.DMA for _ in range(2)],
            [pltpu.SemaphoreType.DMA for _ in range(2)],
            [pltpu.SemaphoreType.DMA for _ in range(2)],
            [pltpu.SemaphoreType.DMA for _ in range(2)],
        ],
    )
    def sc_kernel(nf_hbm, w_hbm, init_hbm, snd_hbm, rcv_hbm, out_hbm,
                  sidx_ld, ridx_ld, ridx_sc, rows, wrows, prod, msg_sh,
                  isem, gsem, wsem, ssem):
        cid = lax.axis_index("c")
        sid = lax.axis_index("s")
        wid = sid * NC + cid
        start = wid * per + lax.min(wid, extra)
        count = per + jnp.where(wid < extra, 1, 0)
        init2d = init_hbm.at[cid] if init_hbm.shape == (NC, N, D) else init_hbm

        def idx_load(i, b):
            @pl.when(i < count)
            def _():
                base = pl.multiple_of(e0 + (start + i) * C, 8)
                pltpu.async_copy(snd_hbm.at[pl.ds(base, C)], sidx_ld[b],
                                 isem[b])
                pltpu.async_copy(rcv_hbm.at[pl.ds(base, C)], ridx_ld[b],
                                 isem[b])

        def idx_wait(i, b):
            @pl.when(i < count)
            def _():
                pltpu.make_async_copy(snd_hbm.at[pl.ds(0, C)], sidx_ld[b],
                                      isem[b]).wait()
                pltpu.make_async_copy(snd_hbm.at[pl.ds(0, C)], sidx_ld[b],
                                      isem[b]).wait()

        def fetch(i, b):
            @pl.when(i < count)
            def _():
                base = pl.multiple_of((start + i) * C, 8)
                pltpu.async_copy(nf_hbm.at[sidx_ld[b]], rows[b], gsem[b])
                pltpu.async_copy(w_hbm.at[pl.ds(base, C)], wrows[b], wsem[b])

        idx_load(jnp.int32(0), 0)
        idx_load(jnp.int32(1), 1)

        r0 = pl.multiple_of(sid * ZR, 8)
        pltpu.sync_copy(init2d.at[pl.ds(r0, ZR)],
                        msg_sh.at[pl.ds(r0, ZR)])
        if TAIL:
            @pl.when(sid == NS - 1)
            def _init_tail():
                pltpu.sync_copy(init2d.at[pl.ds(ZR * NS, TAIL)],
                                msg_sh.at[pl.ds(ZR * NS, TAIL)])
        plsc.subcore_barrier()

        idx_wait(jnp.int32(0), 0)
        fetch(jnp.int32(0), 0)

        def step(i, b):
            @pl.when(i < count)
            def _():
                @pl.when(i >= 2)
                def _wait_prev_scatter():
                    pltpu.make_async_copy(
                        prod[b], msg_sh.at[ridx_sc[b]], ssem[b]).wait()

                pltpu.make_async_copy(nf_hbm.at[sidx_ld[b]], rows[b],
                                      gsem[b]).wait()
                pltpu.make_async_copy(w_hbm.at[pl.ds(0, C)], wrows[b],
                                      wsem[b]).wait()

                for j in range(C // LANES):
                    sl = pl.ds(j * LANES, LANES)
                    ridx_sc[b][sl] = ridx_ld[b][sl]

                idx_wait(i + 1, 1 - b)
                fetch(i + 1, 1 - b)
                idx_load(i + 2, b)

                @plsc.parallel_loop(0, C, unroll=4)
                def _mul(r):
                    for j in range(D // LANES):
                        sl = pl.ds(j * LANES, LANES)
                        prod[b][r, sl] = rows[b][r, sl] * wrows[b][r, sl]

                pltpu.async_copy(prod[b], msg_sh.at[ridx_sc[b]], ssem[b],
                                 add=True)

        def pair(j, _):
            step(2 * j, 0)
            step(2 * j + 1, 1)
            return 0

        lax.fori_loop(0, (iters + 1) // 2, pair, 0)

        for b in range(2):
            pltpu.make_async_copy(prod[b], msg_sh.at[ridx_sc[b]],
                                  ssem[b]).wait()

        plsc.subcore_barrier()
        pltpu.sync_copy(msg_sh.at[pl.ds(r0, ZR)],
                        out_hbm.at[cid, pl.ds(r0, ZR)])
        if TAIL:
            @pl.when(sid == NS - 1)
            def _write_tail():
                pltpu.sync_copy(msg_sh.at[pl.ds(ZR * NS, TAIL)],
                                out_hbm.at[cid, pl.ds(ZR * NS, TAIL)])

    return sc_kernel


def _final_body(*refs):
    p_refs = refs[:-2]
    w2_ref = refs[-2]
    out_ref = refs[-1]
    m = p_refs[0][0] + p_refs[0][1]
    for p in p_refs[1:]:
        m = m + p[0] + p[1]
    out_ref[...] = jnp.dot(
        m, w2_ref[...], preferred_element_type=jnp.float32) * (1.0 / AVG_NEIGH)


def _final_kernel(partials_list, W2, bn):
    _, N, D = partials_list[0].shape
    return pl.pallas_call(
        _final_body,
        grid=(N // bn,),
        in_specs=(
            [pl.BlockSpec((NC, bn, D), lambda i: (0, i, 0))
             for _ in partials_list]
            + [pl.BlockSpec((D, D), lambda i: (0, 0))]),
        out_specs=pl.BlockSpec((bn, D), lambda i: (i, 0)),
        out_shape=jax.ShapeDtypeStruct((N, D), jnp.float32),
    )(*partials_list, W2)


def kernel(node_attrs, node_feats, edge_attrs, edge_feats, senders, receivers,
           W_sc, W1, M0, M1, M2, M3, W2):
    N, D = node_feats.shape
    A = node_attrs.shape[1]
    E, SH = edge_attrs.shape
    HID = M0.shape[1]

    wsc_v = jnp.transpose(W_sc.reshape(D, A, D), (1, 0, 2)).reshape(A * D, D)
    m3_s = jnp.transpose(M3.reshape(HID, D, SH), (2, 0, 1)).reshape(SH * HID, D)
    snd = senders.astype(jnp.int32)
    rcv = receivers.astype(jnp.int32)

    nf2 = _nf2_kernel(node_feats, W1, bn=1000)

    segs = (E // 2, E // 2)
    init = jnp.zeros((N, D), jnp.float32)
    ef_t = edge_feats.T
    ea_t = edge_attrs.T
    e0 = 0
    for e_seg in segs:
        w_s = _edge_kernel(ef_t, ea_t, M0.T, M1.T, M2.T,
                           m3_s.astype(jnp.bfloat16),
                           be=3200, e0=e0, e_seg=e_seg)
        init = _make_sc_kernel(N, D, e_seg, C=64, e0=e0)(
            nf2, w_s, init, snd, rcv)
        e0 += e_seg

    sc = _sc_kernel(node_attrs, node_feats, wsc_v, bn=1000)
    out = _final_kernel([init], W2, bn=1000)
    return (out, sc)
